# baseline (speedup 1.0000x reference)
import functools

import jax
import jax.numpy as jnp
import numpy as np
from jax import lax
from jax.experimental import pallas as pl
from jax.experimental.pallas import tpu as pltpu
from jax.experimental.pallas import tpu_sc as plsc

DIM = 16
NUM_CLASSES = 100000
DELTA = np.float32(1.0) / np.float32(100000.0)
SCALE = np.float32(100000.0)

LANES = 16
TR_MODE = False
RPC = 16
XCOLS = 100
CHUNK = RPC * XCOLS


def _bin_index(xx):
    k0 = (xx * SCALE).astype(jnp.int32)
    kk = k0 + 3
    for off in (2, 1, 0, -1, -2):
        kc = k0 + off
        gv = kc.astype(jnp.float32) * DELTA
        kk = jnp.where(gv >= xx, kc, kk)
    kk = jnp.minimum(jnp.maximum(kk, 0), NUM_CLASSES)
    return kk + 2


def _make_sc_kernel(R, n_rows):
    info = plsc.get_sparse_core_info()
    NC, NS = info.num_cores, info.num_subcores
    NW = NC * NS
    assert R % (NW * RPC) == 0
    rows_per_w = R // NW
    n_chunks = rows_per_w // RPC
    segs = []
    off = 0
    while off < CHUNK:
        seg = min(128, CHUNK - off)
        segs.append((off, seg))
        off += seg
    mesh = plsc.VectorSubcoreMesh(core_axis_name="c", subcore_axis_name="s")

    @functools.partial(
        pl.kernel,
        mesh=mesh,
        out_type=jax.ShapeDtypeStruct((XCOLS, DIM, R), jnp.float32),
        scratch_types=[
            pltpu.VMEM((CHUNK,), jnp.float32),
            pltpu.VMEM((CHUNK,), jnp.int32),
            pltpu.VMEM((CHUNK, DIM), jnp.float32),
            pltpu.VMEM((XCOLS, DIM, RPC), jnp.float32),
            pltpu.SemaphoreType.DMA,
            pltpu.SemaphoreType.DMA,
        ],
        compiler_params=pltpu.CompilerParams(use_tc_tiling_on_sc=False, needs_layout_passes=False),
    )
    def sc_embed(x_hbm, table_hbm, out_hbm, xv, idxv, rowsv, outtv, sem, osem):
        wid = lax.axis_index("s") * NC + lax.axis_index("c")
        row_base = wid * rows_per_w
        lane = lax.iota(jnp.int32, LANES)
        fvecs = [jnp.full((LANES,), f, jnp.int32) for f in range(DIM)]

        def chunk_body(g, carry):
            row0 = row_base + g * RPC
            s0 = row0 * XCOLS
            pltpu.sync_copy(x_hbm.at[pl.ds(s0, CHUNK)], xv)

            def bin_body(b, c2):
                xx = xv[pl.ds(b * LANES, LANES)]
                idxv[pl.ds(b * LANES, LANES)] = _bin_index(xx)
                return c2

            lax.fori_loop(0, CHUNK // LANES, bin_body, 0)

            copies = [
                pltpu.async_copy(
                    table_hbm.at[idxv.at[pl.ds(o, n)]],
                    rowsv.at[pl.ds(o, n)],
                    sem,
                )
                for o, n in segs
            ]

            def tr_body(b, c2):
                p = b * LANES + lane
                mm = p * jnp.int32(5243)
                r = lax.shift_right_logical(mm, jnp.int32(19))
                col = p - r * XCOLS
                for f in range(DIM):
                    vals = plsc.load_gather(rowsv, [p, fvecs[f]])
                    plsc.store_scatter(outtv, [col, fvecs[f], r], vals)
                return c2

            @pl.when(g > 0)
            def _():
                pltpu.make_async_copy(
                    outtv, out_hbm.at[:, :, pl.ds(row0, RPC)], osem
                ).wait()

            for j, (o, n) in enumerate(segs):
                copies[j].wait()
                lax.fori_loop(o // LANES, (o + n) // LANES, tr_body, 0)

            pltpu.async_copy(outtv, out_hbm.at[:, :, pl.ds(row0, RPC)], osem)
            return carry

        lax.fori_loop(0, n_chunks, chunk_body, 0)
        last_row0 = row_base + (n_chunks - 1) * RPC
        pltpu.make_async_copy(
            outtv, out_hbm.at[:, :, pl.ds(last_row0, RPC)], osem
        ).wait()

    return sc_embed


def kernel(x, table):
    R, C = x.shape
    out = _make_sc_kernel(R, table.shape[0])(x.reshape(R * C), table)
    return out.transpose(2, 0, 1)


# speedup vs baseline: 1.0426x; 1.0426x over previous
import functools

import jax
import jax.numpy as jnp
import numpy as np
from jax import lax
from jax.experimental import pallas as pl
from jax.experimental.pallas import tpu as pltpu
from jax.experimental.pallas import tpu_sc as plsc

DIM = 16
NUM_CLASSES = 100000
DELTA = np.float32(1.0) / np.float32(100000.0)
SCALE = np.float32(100000.0)

LANES = 16
TR_MODE = False
RPC = 32
XCOLS = 100
CHUNK = RPC * XCOLS


def _bin_index(xx):
    k0 = (xx * SCALE).astype(jnp.int32)
    kk = k0 + 3
    for off in (2, 1, 0, -1, -2):
        kc = k0 + off
        gv = kc.astype(jnp.float32) * DELTA
        kk = jnp.where(gv >= xx, kc, kk)
    kk = jnp.minimum(jnp.maximum(kk, 0), NUM_CLASSES)
    return kk + 2


def _make_sc_kernel(R, n_rows):
    info = plsc.get_sparse_core_info()
    NC, NS = info.num_cores, info.num_subcores
    NW = NC * NS
    assert R % (NW * RPC) == 0
    rows_per_w = R // NW
    n_chunks = rows_per_w // RPC
    segs = []
    off = 0
    while off < CHUNK:
        seg = min(128, CHUNK - off)
        segs.append((off, seg))
        off += seg
    mesh = plsc.VectorSubcoreMesh(core_axis_name="c", subcore_axis_name="s")

    @functools.partial(
        pl.kernel,
        mesh=mesh,
        out_type=jax.ShapeDtypeStruct((XCOLS, DIM, R), jnp.float32),
        scratch_types=[
            pltpu.VMEM((CHUNK,), jnp.float32),
            pltpu.VMEM((CHUNK,), jnp.int32),
            pltpu.VMEM((CHUNK, DIM), jnp.float32),
            pltpu.VMEM((XCOLS, DIM, RPC), jnp.float32),
            pltpu.SemaphoreType.DMA,
            pltpu.SemaphoreType.DMA,
        ],
        compiler_params=pltpu.CompilerParams(use_tc_tiling_on_sc=False, needs_layout_passes=False),
    )
    def sc_embed(x_hbm, table_hbm, out_hbm, xv, idxv, rowsv, outtv, sem, osem):
        wid = lax.axis_index("s") * NC + lax.axis_index("c")
        row_base = wid * rows_per_w
        lane = lax.iota(jnp.int32, LANES)
        fvecs = [jnp.full((LANES,), f, jnp.int32) for f in range(DIM)]

        def chunk_body(g, carry):
            row0 = row_base + g * RPC
            s0 = row0 * XCOLS
            pltpu.sync_copy(x_hbm.at[pl.ds(s0, CHUNK)], xv)

            def bin_body(b, c2):
                xx = xv[pl.ds(b * LANES, LANES)]
                idxv[pl.ds(b * LANES, LANES)] = _bin_index(xx)
                return c2

            lax.fori_loop(0, CHUNK // LANES, bin_body, 0)

            copies = [
                pltpu.async_copy(
                    table_hbm.at[idxv.at[pl.ds(o, n)]],
                    rowsv.at[pl.ds(o, n)],
                    sem,
                )
                for o, n in segs
            ]

            def tr_body(b, c2):
                p = b * LANES + lane
                mm = p * jnp.int32(5243)
                r = lax.shift_right_logical(mm, jnp.int32(19))
                col = p - r * XCOLS
                for f in range(DIM):
                    vals = plsc.load_gather(rowsv, [p, fvecs[f]])
                    plsc.store_scatter(outtv, [col, fvecs[f], r], vals)
                return c2

            @pl.when(g > 0)
            def _():
                pltpu.make_async_copy(
                    outtv, out_hbm.at[:, :, pl.ds(row0, RPC)], osem
                ).wait()

            for j, (o, n) in enumerate(segs):
                copies[j].wait()
                lax.fori_loop(o // LANES, (o + n) // LANES, tr_body, 0)

            pltpu.async_copy(outtv, out_hbm.at[:, :, pl.ds(row0, RPC)], osem)
            return carry

        lax.fori_loop(0, n_chunks, chunk_body, 0)
        last_row0 = row_base + (n_chunks - 1) * RPC
        pltpu.make_async_copy(
            outtv, out_hbm.at[:, :, pl.ds(last_row0, RPC)], osem
        ).wait()

    return sc_embed


def kernel(x, table):
    R, C = x.shape
    out = _make_sc_kernel(R, table.shape[0])(x.reshape(R * C), table)
    return out.transpose(2, 0, 1)


# speedup vs baseline: 1.0480x; 1.0053x over previous
import functools

import jax
import jax.numpy as jnp
import numpy as np
from jax import lax
from jax.experimental import pallas as pl
from jax.experimental.pallas import tpu as pltpu
from jax.experimental.pallas import tpu_sc as plsc

DIM = 16
NUM_CLASSES = 100000
DELTA = np.float32(1.0) / np.float32(100000.0)
SCALE = np.float32(100000.0)

LANES = 16
XCOLS = 100
NJG = 4
CH_J = XCOLS // NJG
CH_I = 128
CHUNK = CH_I * CH_J
M25 = np.int32(5243)
S25 = np.int32(17)


def _bin_index(xx):
    k0 = (xx * SCALE).astype(jnp.int32)
    kk = k0 + 3
    for off in (2, 1, 0, -1, -2):
        kc = k0 + off
        gv = kc.astype(jnp.float32) * DELTA
        kk = jnp.where(gv >= xx, kc, kk)
    kk = jnp.minimum(jnp.maximum(kk, 0), NUM_CLASSES)
    return kk + 2


def _make_sc_kernel(R, n_rows):
    info = plsc.get_sparse_core_info()
    NC, NS = info.num_cores, info.num_subcores
    NW = NC * NS
    NIG = NW // NJG
    rows_per_w = R // NIG
    n_chunks = rows_per_w // CH_I
    nsegs = CHUNK // 128
    mesh = plsc.VectorSubcoreMesh(core_axis_name="c", subcore_axis_name="s")

    @functools.partial(
        pl.kernel,
        mesh=mesh,
        out_type=jax.ShapeDtypeStruct((XCOLS, DIM, R), jnp.float32),
        scratch_types=[
            pltpu.VMEM((CH_I, XCOLS), jnp.float32),
            pltpu.VMEM((CHUNK,), jnp.int32),
            pltpu.VMEM((CHUNK, DIM), jnp.float32),
            pltpu.VMEM((CH_J, DIM, CH_I), jnp.float32),
            pltpu.SemaphoreType.DMA,
            pltpu.SemaphoreType.DMA,
        ],
        compiler_params=pltpu.CompilerParams(
            use_tc_tiling_on_sc=False, needs_layout_passes=False
        ),
    )
    def sc_embed(x_hbm, table_hbm, out_hbm, xv, idxv, rowsv, outtv, sem, osem):
        wid = lax.axis_index("s") * NC + lax.axis_index("c")
        jg = lax.bitwise_and(wid, NJG - 1)
        ig = lax.shift_right_logical(wid, 2)
        j0 = jg * CH_J
        i_base = ig * rows_per_w
        lane = lax.iota(jnp.int32, LANES)
        fvecs = [jnp.full((LANES,), f, jnp.int32) for f in range(DIM)]
        bmask = lane >= (2 * LANES - CH_J)

        def chunk_body(g, carry):
            i0 = i_base + g * CH_I
            pltpu.sync_copy(x_hbm.at[pl.ds(i0, CH_I)], xv)

            def bin_body(il, c2):
                base = il * CH_J
                xa = xv[il, pl.ds(j0, LANES)]
                plsc.store_scatter(idxv, [base + lane], _bin_index(xa))
                xb = xv[il, pl.ds(j0 + CH_J - LANES, LANES)]
                plsc.store_scatter(
                    idxv,
                    [base + (CH_J - LANES) + lane],
                    _bin_index(xb),
                    mask=bmask,
                )
                return c2

            lax.fori_loop(0, CH_I, bin_body, 0)

            copies = [
                pltpu.async_copy(
                    table_hbm.at[idxv.at[pl.ds(o * 128, 128)]],
                    rowsv.at[pl.ds(o * 128, 128)],
                    sem,
                )
                for o in range(nsegs)
            ]

            def tr_body(b, c2):
                p = b * LANES + lane
                il = lax.shift_right_logical(p * M25, S25)
                jj = p - il * CH_J
                for f in range(DIM):
                    vals = plsc.load_gather(rowsv, [p, fvecs[f]])
                    plsc.store_scatter(outtv, [jj, fvecs[f], il], vals)
                return c2

            @pl.when(g > 0)
            def _():
                pltpu.make_async_copy(
                    outtv,
                    out_hbm.at[pl.ds(j0, CH_J), :, pl.ds(i0, CH_I)],
                    osem,
                ).wait()

            for o in range(nsegs):
                copies[o].wait()
                lax.fori_loop(o * 8, (o + 1) * 8, tr_body, 0)

            pltpu.async_copy(
                outtv, out_hbm.at[pl.ds(j0, CH_J), :, pl.ds(i0, CH_I)], osem
            )
            return carry

        lax.fori_loop(0, n_chunks, chunk_body, 0)
        last_i0 = i_base + (n_chunks - 1) * CH_I
        pltpu.make_async_copy(
            outtv, out_hbm.at[pl.ds(j0, CH_J), :, pl.ds(last_i0, CH_I)], osem
        ).wait()

    return sc_embed


def kernel(x, table):
    R, C = x.shape
    out = _make_sc_kernel(R, table.shape[0])(x, table)
    return out.transpose(2, 0, 1)


# speedup vs baseline: 1.7746x; 1.6932x over previous
import functools

import jax
import jax.numpy as jnp
import numpy as np
from jax import lax
from jax.experimental import pallas as pl
from jax.experimental.pallas import tpu as pltpu
from jax.experimental.pallas import tpu_sc as plsc

DIM = 16
NUM_CLASSES = 100000
DELTA = np.float32(1.0) / np.float32(100000.0)
SCALE = np.float32(100000.0)

LANES = 16
XCOLS = 100
NJG = 4
CH_J = XCOLS // NJG
CH_I = 128
CHUNK = CH_I * CH_J
M25 = np.int32(5243)
S25 = np.int32(17)


def _bin_index(xx):
    k0 = (xx * SCALE).astype(jnp.int32)
    kk = k0 + 3
    for off in (2, 1, 0, -1, -2):
        kc = k0 + off
        gv = kc.astype(jnp.float32) * DELTA
        kk = jnp.where(gv >= xx, kc, kk)
    kk = jnp.minimum(jnp.maximum(kk, 0), NUM_CLASSES)
    return kk + 2


def _make_sc_kernel(R, n_rows):
    info = plsc.get_sparse_core_info()
    NC, NS = info.num_cores, info.num_subcores
    NW = NC * NS
    NIG = NW // NJG
    rows_per_w = R // NIG
    n_chunks = rows_per_w // CH_I
    nsegs = CHUNK // 128
    mesh = plsc.VectorSubcoreMesh(core_axis_name="c", subcore_axis_name="s")

    @functools.partial(
        pl.kernel,
        mesh=mesh,
        out_type=jax.ShapeDtypeStruct((XCOLS, DIM, R), jnp.float32),
        scratch_types=[
            pltpu.VMEM((CH_I, XCOLS), jnp.float32),
            pltpu.VMEM((CHUNK,), jnp.int32),
            pltpu.VMEM((CHUNK, DIM), jnp.float32),
            pltpu.VMEM((CH_J, DIM, CH_I + 1), jnp.float32),
            pltpu.SemaphoreType.DMA,
            pltpu.SemaphoreType.DMA,
        ],
        compiler_params=pltpu.CompilerParams(
            use_tc_tiling_on_sc=False, needs_layout_passes=False
        ),
    )
    def sc_embed(x_hbm, table_hbm, out_hbm, xv, idxv, rowsv, outtv, sem, osem):
        wid = lax.axis_index("s") * NC + lax.axis_index("c")
        jg = lax.bitwise_and(wid, NJG - 1)
        ig = lax.shift_right_logical(wid, 2)
        j0 = jg * CH_J
        i_base = ig * rows_per_w
        lane = lax.iota(jnp.int32, LANES)
        fds = [jnp.bitwise_and(f0 + lane, DIM - 1) for f0 in range(DIM)]
        bmask = lane >= (2 * LANES - CH_J)

        def chunk_body(g, carry):
            i0 = i_base + g * CH_I
            pltpu.sync_copy(x_hbm.at[pl.ds(i0, CH_I)], xv)

            def bin_body(il, c2):
                base = il * CH_J
                xa = xv[il, pl.ds(j0, LANES)]
                plsc.store_scatter(idxv, [base + lane], _bin_index(xa))
                xb = xv[il, pl.ds(j0 + CH_J - LANES, LANES)]
                plsc.store_scatter(
                    idxv,
                    [base + (CH_J - LANES) + lane],
                    _bin_index(xb),
                    mask=bmask,
                )
                return c2

            lax.fori_loop(0, CH_I, bin_body, 0)

            copies = [
                pltpu.async_copy(
                    table_hbm.at[idxv.at[pl.ds(o * 128, 128)]],
                    rowsv.at[pl.ds(o * 128, 128)],
                    sem,
                )
                for o in range(nsegs)
            ]

            def tr_body(b, c2):
                p = b * LANES + lane
                il = lax.shift_right_logical(p * M25, S25)
                jj = p - il * CH_J
                for f0 in range(DIM):
                    vals = plsc.load_gather(rowsv, [p, fds[f0]])
                    plsc.store_scatter(outtv, [jj, fds[f0], il], vals)
                return c2

            @pl.when(g > 0)
            def _():
                pltpu.make_async_copy(
                    outtv.at[:, :, pl.ds(0, CH_I)],
                    out_hbm.at[pl.ds(j0, CH_J), :, pl.ds(i0, CH_I)],
                    osem,
                ).wait()

            for o in range(nsegs):
                copies[o].wait()
                lax.fori_loop(o * 8, (o + 1) * 8, tr_body, 0)

            pltpu.async_copy(
                outtv.at[:, :, pl.ds(0, CH_I)],
                out_hbm.at[pl.ds(j0, CH_J), :, pl.ds(i0, CH_I)],
                osem,
            )
            return carry

        lax.fori_loop(0, n_chunks, chunk_body, 0)
        last_i0 = i_base + (n_chunks - 1) * CH_I
        pltpu.make_async_copy(
            outtv.at[:, :, pl.ds(0, CH_I)],
            out_hbm.at[pl.ds(j0, CH_J), :, pl.ds(last_i0, CH_I)],
            osem,
        ).wait()

    return sc_embed


def kernel(x, table):
    R, C = x.shape
    out = _make_sc_kernel(R, table.shape[0])(x, table)
    return out.transpose(2, 0, 1)


# speedup vs baseline: 1.8382x; 1.0359x over previous
import functools

import jax
import jax.numpy as jnp
import numpy as np
from jax import lax
from jax.experimental import pallas as pl
from jax.experimental.pallas import tpu as pltpu
from jax.experimental.pallas import tpu_sc as plsc

DIM = 16
NUM_CLASSES = 100000
DELTA = np.float32(1.0) / np.float32(100000.0)
SCALE = np.float32(100000.0)

LANES = 16
XCOLS = 100
NJG = 4
CH_J = XCOLS // NJG
CH_I = 128
CHUNK = CH_I * CH_J
M25 = np.int32(5243)
S25 = np.int32(17)


def _bin_index(xx):
    k0 = (xx * SCALE).astype(jnp.int32)
    kk = k0 + 3
    for off in (2, 1, 0, -1):
        kc = k0 + off
        gv = kc.astype(jnp.float32) * DELTA
        kk = jnp.where(gv >= xx, kc, kk)
    kk = jnp.minimum(jnp.maximum(kk, 0), NUM_CLASSES)
    return kk + 2


def _make_sc_kernel(R, n_rows):
    info = plsc.get_sparse_core_info()
    NC, NS = info.num_cores, info.num_subcores
    NW = NC * NS
    NIG = NW // NJG
    rows_per_w = R // NIG
    n_chunks = rows_per_w // CH_I
    nsegs = CHUNK // 128
    mesh = plsc.VectorSubcoreMesh(core_axis_name="c", subcore_axis_name="s")

    @functools.partial(
        pl.kernel,
        mesh=mesh,
        out_type=jax.ShapeDtypeStruct((XCOLS, DIM, R), jnp.float32),
        scratch_types=[
            pltpu.VMEM((CH_I, XCOLS), jnp.float32),
            pltpu.VMEM((CHUNK,), jnp.int32),
            pltpu.VMEM((CHUNK, DIM), jnp.float32),
            pltpu.VMEM((CH_J, DIM, CH_I + 1), jnp.float32),
            pltpu.SemaphoreType.DMA,
            pltpu.SemaphoreType.DMA,
            pltpu.SemaphoreType.DMA,
        ],
        compiler_params=pltpu.CompilerParams(
            use_tc_tiling_on_sc=False, needs_layout_passes=False
        ),
    )
    def sc_embed(x_hbm, table_hbm, out_hbm, xv, idxv, rowsv, outtv, sem, osem, xsem):
        wid = lax.axis_index("s") * NC + lax.axis_index("c")
        jg = lax.bitwise_and(wid, NJG - 1)
        ig = lax.shift_right_logical(wid, 2)
        j0 = jg * CH_J
        i_base = ig * rows_per_w
        lane = lax.iota(jnp.int32, LANES)
        fds = [jnp.bitwise_and(f0 + lane, DIM - 1) for f0 in range(DIM)]
        bmask = lane >= (2 * LANES - CH_J)

        pltpu.async_copy(x_hbm.at[pl.ds(i_base, CH_I)], xv, xsem)

        def chunk_body(g, carry):
            i0 = i_base + g * CH_I
            pltpu.make_async_copy(
                x_hbm.at[pl.ds(i0, CH_I)], xv, xsem
            ).wait()

            def bin_body(il, c2):
                base = il * CH_J
                xa = xv[il, pl.ds(j0, LANES)]
                plsc.store_scatter(idxv, [base + lane], _bin_index(xa))
                xb = xv[il, pl.ds(j0 + CH_J - LANES, LANES)]
                plsc.store_scatter(
                    idxv,
                    [base + (CH_J - LANES) + lane],
                    _bin_index(xb),
                    mask=bmask,
                )
                return c2

            lax.fori_loop(0, CH_I, bin_body, 0)

            i0n = i_base + jnp.minimum(g + 1, n_chunks - 1) * CH_I
            pltpu.async_copy(x_hbm.at[pl.ds(i0n, CH_I)], xv, xsem)

            copies = [
                pltpu.async_copy(
                    table_hbm.at[idxv.at[pl.ds(o * 128, 128)]],
                    rowsv.at[pl.ds(o * 128, 128)],
                    sem,
                )
                for o in range(nsegs)
            ]

            def tr_body(b, c2):
                p = b * LANES + lane
                il = lax.shift_right_logical(p * M25, S25)
                jj = p - il * CH_J
                for f0 in range(DIM):
                    vals = plsc.load_gather(rowsv, [p, fds[f0]])
                    plsc.store_scatter(outtv, [jj, fds[f0], il], vals)
                return c2

            @pl.when(g > 0)
            def _():
                pltpu.make_async_copy(
                    outtv.at[:, :, pl.ds(0, CH_I)],
                    out_hbm.at[pl.ds(j0, CH_J), :, pl.ds(i0, CH_I)],
                    osem,
                ).wait()

            for o in range(nsegs):
                copies[o].wait()
                lax.fori_loop(o * 8, (o + 1) * 8, tr_body, 0)

            pltpu.async_copy(
                outtv.at[:, :, pl.ds(0, CH_I)],
                out_hbm.at[pl.ds(j0, CH_J), :, pl.ds(i0, CH_I)],
                osem,
            )
            return carry

        lax.fori_loop(0, n_chunks, chunk_body, 0)
        pltpu.make_async_copy(
            x_hbm.at[pl.ds(i_base, CH_I)], xv, xsem
        ).wait()
        last_i0 = i_base + (n_chunks - 1) * CH_I
        pltpu.make_async_copy(
            outtv.at[:, :, pl.ds(0, CH_I)],
            out_hbm.at[pl.ds(j0, CH_J), :, pl.ds(last_i0, CH_I)],
            osem,
        ).wait()

    return sc_embed


def kernel(x, table):
    R, C = x.shape
    out = _make_sc_kernel(R, table.shape[0])(x, table)
    return out.transpose(2, 0, 1)


# speedup vs baseline: 2.3059x; 1.2545x over previous
import functools

import jax
import jax.numpy as jnp
import numpy as np
from jax import lax
from jax.experimental import pallas as pl
from jax.experimental.pallas import tpu as pltpu
from jax.experimental.pallas import tpu_sc as plsc

DIM = 16
NUM_CLASSES = 100000
DELTA = np.float32(1.0) / np.float32(100000.0)
SCALE = np.float32(100000.0)

LANES = 16
XCOLS = 100
NJG = 4
CH_J = XCOLS // NJG
CH_I = 128
CHUNK = CH_I * CH_J
M25 = np.int32(5243)
S25 = np.int32(17)


def _bin_index(xx):
    k0 = (xx * SCALE).astype(jnp.int32)
    kk = k0 + 3
    for off in (2, 1, 0, -1):
        kc = k0 + off
        gv = kc.astype(jnp.float32) * DELTA
        kk = jnp.where(gv >= xx, kc, kk)
    kk = jnp.minimum(jnp.maximum(kk, 0), NUM_CLASSES)
    return kk + 2


def _make_sc_kernel(R, n_rows):
    info = plsc.get_sparse_core_info()
    NC, NS = info.num_cores, info.num_subcores
    NW = NC * NS
    NIG = NW // NJG
    rows_per_w = R // NIG
    n_chunks = rows_per_w // CH_I
    nsegs = CHUNK // 128
    mesh = plsc.VectorSubcoreMesh(core_axis_name="c", subcore_axis_name="s")

    @functools.partial(
        pl.kernel,
        mesh=mesh,
        out_type=jax.ShapeDtypeStruct((XCOLS, 2, R // 128, 8, 128), jnp.float32),
        scratch_types=[
            pltpu.VMEM((CH_I, XCOLS), jnp.float32),
            pltpu.VMEM((CHUNK,), jnp.int32),
            pltpu.VMEM((CHUNK, DIM), jnp.float32),
            pltpu.VMEM((CH_J, 2, 8, CH_I), jnp.float32),
            pltpu.SemaphoreType.DMA,
            pltpu.SemaphoreType.DMA,
            pltpu.SemaphoreType.DMA,
        ],
        compiler_params=pltpu.CompilerParams(
            use_tc_tiling_on_sc=False, needs_layout_passes=False
        ),
    )
    def sc_embed(x_hbm, table_hbm, out_hbm, xv, idxv, rowsv, outtv, sem, osem, xsem):
        wid = lax.axis_index("s") * NC + lax.axis_index("c")
        jg = lax.bitwise_and(wid, NJG - 1)
        ig = lax.shift_right_logical(wid, 2)
        j0 = jg * CH_J
        i_base = ig * rows_per_w
        lane = lax.iota(jnp.int32, LANES)
        fds = [jnp.bitwise_and(f0 + lane, DIM - 1) for f0 in range(DIM)]
        tfs = [lax.shift_right_logical(fd, 3) for fd in fds]
        frs = [jnp.bitwise_and(fd, 7) for fd in fds]
        lane25 = lane * CH_J
        bmask = lane >= (2 * LANES - CH_J)

        pltpu.async_copy(x_hbm.at[pl.ds(i_base, CH_I)], xv, xsem)

        def chunk_body(g, carry):
            i0 = i_base + g * CH_I
            pltpu.make_async_copy(
                x_hbm.at[pl.ds(i0, CH_I)], xv, xsem
            ).wait()

            def bin_body(il, c2):
                base = il * CH_J
                xa = xv[il, pl.ds(j0, LANES)]
                plsc.store_scatter(idxv, [base + lane], _bin_index(xa))
                xb = xv[il, pl.ds(j0 + CH_J - LANES, LANES)]
                plsc.store_scatter(
                    idxv,
                    [base + (CH_J - LANES) + lane],
                    _bin_index(xb),
                    mask=bmask,
                )
                return c2

            lax.fori_loop(0, CH_I, bin_body, 0)

            i0n = i_base + jnp.minimum(g + 1, n_chunks - 1) * CH_I
            pltpu.async_copy(x_hbm.at[pl.ds(i0n, CH_I)], xv, xsem)

            copies = [
                pltpu.async_copy(
                    table_hbm.at[idxv.at[pl.ds(o * 128, 128)]],
                    rowsv.at[pl.ds(o * 128, 128)],
                    sem,
                )
                for o in range(nsegs)
            ]

            def tr_body(jj, c2):
                jv = jnp.full((LANES,), jj, jnp.int32)

                def tr_inner(ilb, c3):
                    pv = (ilb * (LANES * CH_J) + jj) + lane25
                    ilv = ilb * LANES + lane
                    for f0 in range(DIM):
                        vals = plsc.load_gather(rowsv, [pv, fds[f0]])
                        plsc.store_scatter(
                            outtv, [jv, tfs[f0], frs[f0], ilv], vals
                        )
                    return c3

                lax.fori_loop(0, CH_I // LANES, tr_inner, 0)
                return c2

            @pl.when(g > 0)
            def _():
                pltpu.make_async_copy(
                    outtv,
                    out_hbm.at[pl.ds(j0, CH_J), :, i0 // CH_I, :, :],
                    osem,
                ).wait()

            for c in copies:
                c.wait()
            lax.fori_loop(0, CH_J, tr_body, 0)

            pltpu.async_copy(
                outtv,
                out_hbm.at[pl.ds(j0, CH_J), :, i0 // CH_I, :, :],
                osem,
            )
            return carry

        lax.fori_loop(0, n_chunks, chunk_body, 0)
        pltpu.make_async_copy(
            x_hbm.at[pl.ds(i_base, CH_I)], xv, xsem
        ).wait()
        last_i0 = i_base + (n_chunks - 1) * CH_I
        pltpu.make_async_copy(
            outtv,
            out_hbm.at[pl.ds(j0, CH_J), :, last_i0 // CH_I, :, :],
            osem,
        ).wait()

    return sc_embed


def kernel(x, table):
    R, C = x.shape
    out = _make_sc_kernel(R, table.shape[0])(x, table)
    return out.transpose(2, 4, 0, 1, 3).reshape(R, C, DIM)


# speedup vs baseline: 2.6013x; 1.1281x over previous
import functools

import jax
import jax.numpy as jnp
import numpy as np
from jax import lax
from jax.experimental import pallas as pl
from jax.experimental.pallas import tpu as pltpu
from jax.experimental.pallas import tpu_sc as plsc

DIM = 16
NUM_CLASSES = 100000
DELTA = np.float32(1.0) / np.float32(100000.0)
SCALE = np.float32(100000.0)

LANES = 16
XCOLS = 100
NJG = 4
CH_J = XCOLS // NJG
CH_I = 128
CHUNK = CH_I * CH_J
M25 = np.int32(5243)
S25 = np.int32(17)


def _bin_index(xx):
    k0 = (xx * SCALE).astype(jnp.int32)
    kk = k0 + 3
    for off in (2, 1, 0, -1):
        kc = k0 + off
        gv = kc.astype(jnp.float32) * DELTA
        kk = jnp.where(gv >= xx, kc, kk)
    kk = jnp.minimum(jnp.maximum(kk, 0), NUM_CLASSES)
    return kk + 2


def _make_sc_kernel(R, n_rows):
    info = plsc.get_sparse_core_info()
    NC, NS = info.num_cores, info.num_subcores
    NW = NC * NS
    NIG = NW // NJG
    rows_per_w = R // NIG
    n_chunks = rows_per_w // CH_I
    nsegs = CHUNK // 128
    mesh = plsc.VectorSubcoreMesh(core_axis_name="c", subcore_axis_name="s")

    @functools.partial(
        pl.kernel,
        mesh=mesh,
        out_type=jax.ShapeDtypeStruct((XCOLS, 2, R // 128, 8, 128), jnp.float32),
        scratch_types=[
            pltpu.VMEM((CH_I, XCOLS), jnp.float32),
            pltpu.VMEM((CHUNK,), jnp.int32),
            pltpu.VMEM((CHUNK, DIM), jnp.float32),
            pltpu.VMEM((CH_J, 2, 8, CH_I), jnp.float32),
            pltpu.SemaphoreType.DMA,
            pltpu.SemaphoreType.DMA,
            pltpu.SemaphoreType.DMA,
        ],
        compiler_params=pltpu.CompilerParams(
            use_tc_tiling_on_sc=False, needs_layout_passes=False
        ),
    )
    def sc_embed(x_hbm, table_hbm, out_hbm, xv, idxv, rowsv, outtv, sem, osem, xsem):
        wid = lax.axis_index("s") * NC + lax.axis_index("c")
        jg = lax.bitwise_and(wid, NJG - 1)
        ig = lax.shift_right_logical(wid, 2)
        j0 = jg * CH_J
        i_base = ig * rows_per_w
        lane = lax.iota(jnp.int32, LANES)
        fds = [jnp.bitwise_and(f0 + lane, DIM - 1) for f0 in range(DIM)]
        tfs = [lax.shift_right_logical(fd, 3) for fd in fds]
        frs = [jnp.bitwise_and(fd, 7) for fd in fds]
        lane25 = lane * CH_J
        bmask = lane >= (2 * LANES - CH_J)

        pltpu.async_copy(x_hbm.at[pl.ds(i_base, CH_I)], xv, xsem)

        def chunk_body(g, carry):
            i0 = i_base + g * CH_I
            pltpu.make_async_copy(
                x_hbm.at[pl.ds(i0, CH_I)], xv, xsem
            ).wait()

            def bin_body(il, c2):
                base = il * CH_J
                xa = xv[il, pl.ds(j0, LANES)]
                plsc.store_scatter(idxv, [base + lane], _bin_index(xa))
                xb = xv[il, pl.ds(j0 + CH_J - LANES, LANES)]
                plsc.store_scatter(
                    idxv,
                    [base + (CH_J - LANES) + lane],
                    _bin_index(xb),
                    mask=bmask,
                )
                return c2

            GRP = 5
            copies = []
            il_done = 0
            for grp in range(nsegs // GRP):
                hi = 128 * GRP * (grp + 1)
                il_need = (hi - 1) // CH_J + 1
                lax.fori_loop(il_done, il_need, bin_body, 0)
                il_done = il_need
                for o in range(GRP * grp, GRP * (grp + 1)):
                    copies.append(
                        pltpu.async_copy(
                            table_hbm.at[idxv.at[pl.ds(o * 128, 128)]],
                            rowsv.at[pl.ds(o * 128, 128)],
                            sem,
                        )
                    )

            i0n = i_base + jnp.minimum(g + 1, n_chunks - 1) * CH_I
            pltpu.async_copy(x_hbm.at[pl.ds(i0n, CH_I)], xv, xsem)

            def tr_outer(ilb, c2):
                ilv = ilb * LANES + lane

                def tr_jj(jj, c3):
                    jv = jnp.full((LANES,), jj, jnp.int32)
                    pv = (ilb * (LANES * CH_J) + jj) + lane25
                    for f0 in range(DIM):
                        vals = plsc.load_gather(rowsv, [pv, fds[f0]])
                        plsc.store_scatter(
                            outtv, [jv, tfs[f0], frs[f0], ilv], vals
                        )
                    return c3

                lax.fori_loop(0, CH_J, tr_jj, 0)
                return c2

            @pl.when(g > 0)
            def _():
                pltpu.make_async_copy(
                    outtv,
                    out_hbm.at[pl.ds(j0, CH_J), :, i0 // CH_I, :, :],
                    osem,
                ).wait()

            half = CH_I // LANES // 2
            need_a = (half * LANES * CH_J - 1) // 128 + 1
            for k in range(need_a):
                copies[k].wait()
            lax.fori_loop(0, half, tr_outer, 0)
            for k in range(need_a, nsegs):
                copies[k].wait()
            lax.fori_loop(half, CH_I // LANES, tr_outer, 0)

            pltpu.async_copy(
                outtv,
                out_hbm.at[pl.ds(j0, CH_J), :, i0 // CH_I, :, :],
                osem,
            )
            return carry

        lax.fori_loop(0, n_chunks, chunk_body, 0)
        pltpu.make_async_copy(
            x_hbm.at[pl.ds(i_base, CH_I)], xv, xsem
        ).wait()
        last_i0 = i_base + (n_chunks - 1) * CH_I
        pltpu.make_async_copy(
            outtv,
            out_hbm.at[pl.ds(j0, CH_J), :, last_i0 // CH_I, :, :],
            osem,
        ).wait()

    return sc_embed


def kernel(x, table):
    R, C = x.shape
    out = _make_sc_kernel(R, table.shape[0])(x, table)
    return out.transpose(2, 4, 0, 1, 3).reshape(R, C, DIM)


# speedup vs baseline: 2.7587x; 1.0605x over previous
import functools

import jax
import jax.numpy as jnp
import numpy as np
from jax import lax
from jax.experimental import pallas as pl
from jax.experimental.pallas import tpu as pltpu
from jax.experimental.pallas import tpu_sc as plsc

DIM = 16
NUM_CLASSES = 100000
DELTA = np.float32(1.0) / np.float32(100000.0)
SCALE = np.float32(100000.0)

LANES = 16
XCOLS = 100
NJG = 4
CH_J = XCOLS // NJG
CH_I = 128
CHUNK = CH_I * CH_J
M25 = np.int32(5243)
S25 = np.int32(17)


def _bin_index(xx):
    k0 = (xx * SCALE).astype(jnp.int32)
    kk = k0 + 3
    for off in (2, 1, 0, -1):
        kc = k0 + off
        gv = kc.astype(jnp.float32) * DELTA
        kk = jnp.where(gv >= xx, kc, kk)
    kk = jnp.minimum(jnp.maximum(kk, 0), NUM_CLASSES)
    return kk + 2


def _make_sc_kernel(R, n_rows):
    info = plsc.get_sparse_core_info()
    NC, NS = info.num_cores, info.num_subcores
    NW = NC * NS
    NIG = NW // NJG
    rows_per_w = R // NIG
    n_chunks = rows_per_w // CH_I
    nsegs = CHUNK // 128
    mesh = plsc.VectorSubcoreMesh(core_axis_name="c", subcore_axis_name="s")

    @functools.partial(
        pl.kernel,
        mesh=mesh,
        out_type=jax.ShapeDtypeStruct((XCOLS, 2, R // 128, 8, 128), jnp.float32),
        scratch_types=[
            pltpu.VMEM((CH_J, CH_I), jnp.float32),
            pltpu.VMEM((CHUNK,), jnp.int32),
            pltpu.VMEM((CHUNK, DIM), jnp.float32),
            pltpu.VMEM((CH_J, 2, 8, CH_I), jnp.float32),
            pltpu.SemaphoreType.DMA,
            pltpu.SemaphoreType.DMA,
            pltpu.SemaphoreType.DMA,
        ],
        compiler_params=pltpu.CompilerParams(
            use_tc_tiling_on_sc=False, needs_layout_passes=False
        ),
    )
    def sc_embed(x_hbm, table_hbm, out_hbm, xv, idxv, rowsv, outtv, sem, osem, xsem):
        wid = lax.axis_index("s") * NC + lax.axis_index("c")
        jg = lax.bitwise_and(wid, NJG - 1)
        ig = lax.shift_right_logical(wid, 2)
        j0 = jg * CH_J
        i_base = ig * rows_per_w
        lane = lax.iota(jnp.int32, LANES)
        fds = [jnp.bitwise_and(f0 + lane, DIM - 1) for f0 in range(DIM)]
        tfs = [lax.shift_right_logical(fd, 3) for fd in fds]
        frs = [jnp.bitwise_and(fd, 7) for fd in fds]
        lane25 = lane * CH_J

        pltpu.async_copy(
            x_hbm.at[pl.ds(j0, CH_J), pl.ds(i_base, CH_I)], xv, xsem
        )

        def chunk_body(g, carry):
            i0 = i_base + g * CH_I
            pltpu.make_async_copy(
                x_hbm.at[pl.ds(j0, CH_J), pl.ds(i0, CH_I)], xv, xsem
            ).wait()

            def make_bin(ib):
                def bin_body(jj, c2):
                    xx = xv[jj, pl.ds(ib * LANES, LANES)]
                    pv = (ib * (LANES * CH_J) + jj) + lane25
                    plsc.store_scatter(idxv, [pv], _bin_index(xx))
                    return c2

                return bin_body

            copies = []
            fired = 0
            for ib in range(CH_I // LANES):
                lax.fori_loop(0, CH_J, make_bin(ib), 0)
                can_fire = (LANES * CH_J * (ib + 1)) // 128
                for o in range(fired, can_fire):
                    copies.append(
                        pltpu.async_copy(
                            table_hbm.at[idxv.at[pl.ds(o * 128, 128)]],
                            rowsv.at[pl.ds(o * 128, 128)],
                            sem,
                        )
                    )
                fired = can_fire

            i0n = i_base + jnp.minimum(g + 1, n_chunks - 1) * CH_I
            pltpu.async_copy(
                x_hbm.at[pl.ds(j0, CH_J), pl.ds(i0n, CH_I)], xv, xsem
            )

            def tr_outer(ilb, c2):
                ilv = ilb * LANES + lane

                def tr_jj(jj, c3):
                    jv = jnp.full((LANES,), jj, jnp.int32)
                    pv = (ilb * (LANES * CH_J) + jj) + lane25
                    for f0 in range(DIM):
                        vals = plsc.load_gather(rowsv, [pv, fds[f0]])
                        plsc.store_scatter(
                            outtv, [jv, tfs[f0], frs[f0], ilv], vals
                        )
                    return c3

                lax.fori_loop(0, CH_J, tr_jj, 0)
                return c2

            @pl.when(g > 0)
            def _():
                pltpu.make_async_copy(
                    outtv,
                    out_hbm.at[pl.ds(j0, CH_J), :, i0 // CH_I, :, :],
                    osem,
                ).wait()

            half = CH_I // LANES // 2
            need_a = (half * LANES * CH_J - 1) // 128 + 1
            for k in range(need_a):
                copies[k].wait()
            lax.fori_loop(0, half, tr_outer, 0)
            for k in range(need_a, nsegs):
                copies[k].wait()
            lax.fori_loop(half, CH_I // LANES, tr_outer, 0)

            pltpu.async_copy(
                outtv,
                out_hbm.at[pl.ds(j0, CH_J), :, i0 // CH_I, :, :],
                osem,
            )
            return carry

        lax.fori_loop(0, n_chunks, chunk_body, 0)
        pltpu.make_async_copy(
            x_hbm.at[pl.ds(j0, CH_J), pl.ds(i_base, CH_I)], xv, xsem
        ).wait()
        last_i0 = i_base + (n_chunks - 1) * CH_I
        pltpu.make_async_copy(
            outtv,
            out_hbm.at[pl.ds(j0, CH_J), :, last_i0 // CH_I, :, :],
            osem,
        ).wait()

    return sc_embed


def kernel(x, table):
    R, C = x.shape
    out = _make_sc_kernel(R, table.shape[0])(x.T, table)
    return out.transpose(2, 4, 0, 1, 3).reshape(R, C, DIM)


# speedup vs baseline: 2.7591x; 1.0001x over previous
import functools

import jax
import jax.numpy as jnp
import numpy as np
from jax import lax
from jax.experimental import pallas as pl
from jax.experimental.pallas import tpu as pltpu
from jax.experimental.pallas import tpu_sc as plsc

DIM = 16
NUM_CLASSES = 100000
DELTA = np.float32(1.0) / np.float32(100000.0)
SCALE = np.float32(100000.0)

LANES = 16
XCOLS = 100
NJG = 4
CH_J = XCOLS // NJG
CH_I = 128
CHUNK = CH_I * CH_J


def _bin_index(xx):
    k0 = (xx * SCALE).astype(jnp.int32)
    kk = k0 + 3
    for off in (2, 1, 0, -1):
        kc = k0 + off
        gv = kc.astype(jnp.float32) * DELTA
        kk = jnp.where(gv >= xx, kc, kk)
    kk = jnp.minimum(jnp.maximum(kk, 0), NUM_CLASSES)
    return kk + 2


def _make_sc_kernel(R, n_rows):
    info = plsc.get_sparse_core_info()
    NC, NS = info.num_cores, info.num_subcores
    NW = NC * NS
    NIG = NW // NJG
    rows_per_w = R // NIG
    n_chunks = rows_per_w // CH_I
    nsegs = CHUNK // 128
    mesh = plsc.VectorSubcoreMesh(core_axis_name="c", subcore_axis_name="s")

    @functools.partial(
        pl.kernel,
        mesh=mesh,
        out_type=jax.ShapeDtypeStruct((XCOLS, 2, R // 128, 8, 128), jnp.float32),
        scratch_types=[
            pltpu.VMEM((CH_J, CH_I), jnp.float32),
            pltpu.VMEM((CHUNK,), jnp.int32),
            pltpu.VMEM((CHUNK, DIM), jnp.float32),
            pltpu.VMEM((CH_J, 2, 8, CH_I), jnp.float32),
            pltpu.SemaphoreType.DMA,
            pltpu.SemaphoreType.DMA,
            pltpu.SemaphoreType.DMA,
        ],
        compiler_params=pltpu.CompilerParams(
            use_tc_tiling_on_sc=False, needs_layout_passes=False
        ),
    )
    def sc_embed(x_hbm, table_hbm, out_hbm, xv, idxv, rowsv, outtv, sem, osem, xsem):
        wid = lax.axis_index("s") * NC + lax.axis_index("c")
        jg = lax.bitwise_and(wid, NJG - 1)
        ig = lax.shift_right_logical(wid, 2)
        j0 = jg * CH_J
        i_base = ig * rows_per_w
        lane = lax.iota(jnp.int32, LANES)
        fds = [jnp.bitwise_and(f0 + lane, DIM - 1) for f0 in range(DIM)]
        tfs = [lax.shift_right_logical(fd, 3) for fd in fds]
        frs = [jnp.bitwise_and(fd, 7) for fd in fds]
        lane25 = lane * CH_J

        pltpu.async_copy(
            x_hbm.at[pl.ds(j0, CH_J), pl.ds(i_base, CH_I)], xv, xsem
        )

        def chunk_body(g, carry):
            i0 = i_base + g * CH_I
            pltpu.make_async_copy(
                x_hbm.at[pl.ds(j0, CH_J), pl.ds(i0, CH_I)], xv, xsem
            ).wait()

            def make_bin(ib):
                def bin_body(jj, c2):
                    xx = xv[jj, pl.ds(ib * LANES, LANES)]
                    pv = (ib * (LANES * CH_J) + jj) + lane25
                    plsc.store_scatter(idxv, [pv], _bin_index(xx))
                    return c2

                return bin_body

            copies = []
            fired = 0
            for ib in range(CH_I // LANES):
                lax.fori_loop(0, CH_J, make_bin(ib), 0)
                can_fire = (LANES * CH_J * (ib + 1)) // 128
                for o in range(fired, can_fire):
                    copies.append(
                        pltpu.async_copy(
                            table_hbm.at[idxv.at[pl.ds(o * 128, 128)]],
                            rowsv.at[pl.ds(o * 128, 128)],
                            sem,
                        )
                    )
                fired = can_fire

            i0n = i_base + jnp.minimum(g + 1, n_chunks - 1) * CH_I
            pltpu.async_copy(
                x_hbm.at[pl.ds(j0, CH_J), pl.ds(i0n, CH_I)], xv, xsem
            )

            def tr_outer(ilb, c2):
                ilv = ilb * LANES + lane

                def tr_jj(jj, c3):
                    jv = jnp.full((LANES,), jj, jnp.int32)
                    pv = (ilb * (LANES * CH_J) + jj) + lane25
                    for f0 in range(DIM):
                        vals = plsc.load_gather(rowsv, [pv, fds[f0]])
                        plsc.store_scatter(
                            outtv, [jv, tfs[f0], frs[f0], ilv], vals
                        )
                    return c3

                lax.fori_loop(0, CH_J, tr_jj, 0)
                return c2

            @pl.when(g > 0)
            def _():
                pltpu.make_async_copy(
                    outtv,
                    out_hbm.at[pl.ds(j0, CH_J), :, i0 // CH_I, :, :],
                    osem,
                ).wait()

            half = CH_I // LANES // 2
            need_a = (half * LANES * CH_J - 1) // 128 + 1
            for k in range(need_a):
                copies[k].wait()
            lax.fori_loop(0, half, tr_outer, 0)
            for k in range(need_a, nsegs):
                copies[k].wait()
            lax.fori_loop(half, CH_I // LANES, tr_outer, 0)

            pltpu.async_copy(
                outtv,
                out_hbm.at[pl.ds(j0, CH_J), :, i0 // CH_I, :, :],
                osem,
            )
            return carry

        lax.fori_loop(0, n_chunks, chunk_body, 0)
        pltpu.make_async_copy(
            x_hbm.at[pl.ds(j0, CH_J), pl.ds(i_base, CH_I)], xv, xsem
        ).wait()
        last_i0 = i_base + (n_chunks - 1) * CH_I
        pltpu.make_async_copy(
            outtv,
            out_hbm.at[pl.ds(j0, CH_J), :, last_i0 // CH_I, :, :],
            osem,
        ).wait()

    return sc_embed


def kernel(x, table):
    R, C = x.shape
    out = _make_sc_kernel(R, table.shape[0])(x.T, table)
    return out.transpose(2, 4, 0, 1, 3).reshape(R, C, DIM)
